# Initial kernel scaffold; baseline (speedup 1.0000x reference)
import jax
import jax.numpy as jnp
from jax.experimental import pallas as pl


def kernel(x, edge_index, edge_attr, batch, global_x, params):
    raise NotImplementedError("write your pallas kernel here")


# speedup vs baseline: 1.9260x; 1.9260x over previous
import functools

import jax
import jax.numpy as jnp
from jax import lax
from jax.experimental import pallas as pl
from jax.experimental.pallas import tpu as pltpu
from jax.experimental.pallas import tpu_sc as plsc

_N = 10000
_E = 320000
_D = 128

_NC = 2
_NS = 16
_NW = _NC * _NS
_EPT = _E // _NW
_C = 80
_NCH = _EPT // _C
_RB = 80
_NRB = _N // _RB

_SC_MESH = plsc.VectorSubcoreMesh(core_axis_name="c", subcore_axis_name="s")


@functools.partial(
    pl.kernel,
    out_type=jax.ShapeDtypeStruct((2 * _N, _D), jnp.float32),
    mesh=_SC_MESH,
    scratch_types=[
        pltpu.VMEM((_C,), jnp.int32),
        pltpu.VMEM((_C,), jnp.int32),
        pltpu.VMEM((_C, _D), jnp.float32),
        pltpu.VMEM((_C, _D), jnp.float32),
        pltpu.VMEM((_RB, _D), jnp.float32),
        pltpu.VMEM_SHARED((_N, _D), jnp.float32),
        pltpu.SemaphoreType.DMA,
        pltpu.SemaphoreType.DMA,
    ],
)
def _sc_edge(h_hbm, e_hbm, src_hbm, dst_hbm, out_hbm,
             sidx, didx, rows, ebuf, zbuf, agg, gsem, esem):
    c = lax.axis_index("c")
    s = lax.axis_index("s")
    wid = s * _NC + c

    blo = s * _NRB // _NS
    bhi = (s + 1) * _NRB // _NS

    def _z(i, t):
        r = i // 8
        j = i - r * 8
        zbuf[r, pl.ds(j * 16, 16)] = jnp.zeros((16,), jnp.float32)
        return t
    lax.fori_loop(0, _RB * 8, _z, 0)

    def _zb(b, t):
        pltpu.sync_copy(zbuf, agg.at[pl.ds(b * _RB, _RB)])
        return t
    lax.fori_loop(blo, bhi, _zb, 0)
    plsc.subcore_barrier()

    def _chunk(n, t):
        base = wid * _EPT + n * _C
        pltpu.sync_copy(src_hbm.at[pl.ds(base, _C)], sidx)
        pltpu.sync_copy(dst_hbm.at[pl.ds(base, _C)], didx)
        cg = pltpu.async_copy(h_hbm.at[sidx], rows, gsem)
        ce = pltpu.async_copy(e_hbm.at[pl.ds(base, _C)], ebuf, esem)
        cg.wait()
        ce.wait()

        def _v(q, u):
            r = q // 8
            j = q - r * 8
            sl = pl.ds(j * 16, 16)
            rows[r, sl] = jnp.maximum(rows[r, sl] + ebuf[r, sl], 0.0)
            return u
        lax.fori_loop(0, _C * 8, _v, 0)

        pltpu.sync_copy(rows, agg.at[didx], add=True)
        return t
    lax.fori_loop(0, _NCH, _chunk, 0)

    plsc.subcore_barrier()

    def _wb(b, t):
        pltpu.sync_copy(agg.at[pl.ds(b * _RB, _RB)],
                        out_hbm.at[pl.ds(c * _N + b * _RB, _RB)])
        return t
    lax.fori_loop(blo, bhi, _wb, 0)


def _mm_bias_body(a_ref, w_ref, b_ref, o_ref):
    o_ref[...] = (jnp.dot(a_ref[...], w_ref[...],
                          preferred_element_type=jnp.float32) + b_ref[...])


def _mm_bias(a, w, b, blk):
    m, k = a.shape
    n = w.shape[1]
    return pl.pallas_call(
        _mm_bias_body,
        grid=(m // blk,),
        in_specs=[pl.BlockSpec((blk, k), lambda i: (i, 0)),
                  pl.BlockSpec((k, n), lambda i: (0, 0)),
                  pl.BlockSpec((1, n), lambda i: (0, 0))],
        out_specs=pl.BlockSpec((blk, n), lambda i: (i, 0)),
        out_shape=jax.ShapeDtypeStruct((m, n), jnp.float32),
    )(a, w, b.reshape(1, n))


def _mlp_body(h_ref, p_ref, w1_ref, b1_ref, w2_ref, b2_ref, gs_ref, bt_ref,
              eps_ref, o_ref):
    h = h_ref[...]
    a = h * (1.0 + eps_ref[0, 0]) + p_ref[0] + p_ref[1]
    t = jnp.maximum(jnp.dot(a, w1_ref[...],
                            preferred_element_type=jnp.float32) + b1_ref[...],
                    0.0)
    t = jnp.dot(t, w2_ref[...], preferred_element_type=jnp.float32) + b2_ref[...]
    t = t * gs_ref[...] + bt_ref[...]
    o_ref[...] = jnp.maximum(t, 0.0) + h


def _mlp(h, parts, w1, b1, w2, b2, gscale, beta, eps, blk=2000):
    full = lambda i: (0, 0)
    return pl.pallas_call(
        _mlp_body,
        grid=(_N // blk,),
        in_specs=[pl.BlockSpec((blk, _D), lambda i: (i, 0)),
                  pl.BlockSpec((2, blk, _D), lambda i: (0, i, 0)),
                  pl.BlockSpec((_D, _D), full),
                  pl.BlockSpec((1, _D), full),
                  pl.BlockSpec((_D, _D), full),
                  pl.BlockSpec((1, _D), full),
                  pl.BlockSpec((1, _D), full),
                  pl.BlockSpec((1, _D), full),
                  pl.BlockSpec((1, 1), full)],
        out_specs=pl.BlockSpec((blk, _D), lambda i: (i, 0)),
        out_shape=jax.ShapeDtypeStruct((_N, _D), jnp.float32),
    )(h, parts, w1, b1.reshape(1, _D), w2, b2.reshape(1, _D),
      gscale.reshape(1, _D), beta.reshape(1, _D), eps.reshape(1, 1))


def _head_body(h_ref, b_ref, gx_ref, wg_ref, bg_ref, wc_ref, bc_ref, o_ref):
    g_iota = lax.broadcasted_iota(jnp.int32, (1, 16), 1)
    oh = (b_ref[...] == g_iota).astype(jnp.float32)
    dn = (((0,), (0,)), ((), ()))
    sums = lax.dot_general(oh, h_ref[...], dn,
                           preferred_element_type=jnp.float32)
    ones = jnp.ones((_N, 1), jnp.float32)
    counts = lax.dot_general(oh, ones, dn,
                             preferred_element_type=jnp.float32)
    pooled = sums / jnp.maximum(counts, 1.0)
    g = jnp.maximum(jnp.dot(gx_ref[...], wg_ref[...],
                            preferred_element_type=jnp.float32) + bg_ref[...],
                    0.0)
    out = (jnp.dot(pooled, wc_ref[0:_D, :],
                   preferred_element_type=jnp.float32)
           + jnp.dot(g, wc_ref[_D:_D + 32, :],
                     preferred_element_type=jnp.float32)
           + bc_ref[...])
    o_ref[...] = out


def _head(h, batch2d, gx, wg, bg, wc, bc):
    return pl.pallas_call(
        _head_body,
        out_shape=jax.ShapeDtypeStruct((16, _D), jnp.float32),
    )(h, batch2d, gx, wg, bg.reshape(1, -1), wc, bc.reshape(1, -1))


def kernel(x, edge_index, edge_attr, batch, global_x, params):
    src = edge_index[0]
    dst = edge_index[1]
    h = _mm_bias(x, params['W_in'], params['b_in'], 2000)
    e = _mm_bias(edge_attr, params['W_e'], params['b_e'], 8000)
    bn_scale = 1.0 / jnp.sqrt(jnp.float32(1.0 + 1e-5))
    for i in range(3):
        p = params['layers'][i]
        parts = _sc_edge(h, e, src, dst).reshape(2, _N, _D)
        h = _mlp(h, parts, p['W1'], p['b1'], p['W2'], p['b2'],
                 p['gamma'] * bn_scale, p['beta'], p['eps'])
    return _head(h, batch.reshape(_N, 1), global_x,
                 params['Wg'], params['bg'], params['Wc'], params['bc'])


# speedup vs baseline: 5.7825x; 3.0022x over previous
import functools

import jax
import jax.numpy as jnp
from jax import lax
from jax.experimental import pallas as pl
from jax.experimental.pallas import tpu as pltpu
from jax.experimental.pallas import tpu_sc as plsc

_N = 10000
_E = 320000
_D = 128

_NC = 2
_NS = 16
_NW = _NC * _NS
_EPT = _E // _NW
_C = 40
_NCH = _EPT // _C
_RB = 80
_NRB = _N // _RB
_ZB = 8

_SC_MESH = plsc.VectorSubcoreMesh(core_axis_name="c", subcore_axis_name="s")
_NBUF = 3


@functools.partial(
    pl.kernel,
    out_type=jax.ShapeDtypeStruct((2 * _N, _D), jnp.float32),
    mesh=_SC_MESH,
    scratch_types=(
        [pltpu.VMEM((_C,), jnp.int32)] * (2 * _NBUF)
        + [pltpu.VMEM((_C, _D), jnp.float32)] * (2 * _NBUF)
        + [pltpu.VMEM((_ZB, _D), jnp.float32)]
        + [pltpu.VMEM_SHARED((_N, _D), jnp.float32)]
        + [pltpu.SemaphoreType.DMA] * (3 * _NBUF)
    ),
)
def _sc_edge(h_hbm, e_hbm, src_hbm, dst_hbm, out_hbm,
             si0, si1, si2, di0, di1, di2,
             r0, r1, r2, e0, e1, e2, zbuf, agg,
             i0, i1, i2, d0, d1, d2, s0, s1, s2):
    sidx = (si0, si1, si2)
    didx = (di0, di1, di2)
    rows = (r0, r1, r2)
    ebuf = (e0, e1, e2)
    isem = (i0, i1, i2)
    dsem = (d0, d1, d2)
    ssem = (s0, s1, s2)
    c = lax.axis_index("c")
    s = lax.axis_index("s")
    wid = s * _NC + c

    blo = s * _NRB // _NS * (_RB // _ZB)
    bhi = (s + 1) * _NRB // _NS * (_RB // _ZB)

    def _z(i, t):
        for j in range(_D // 16):
            zbuf[i, pl.ds(j * 16, 16)] = jnp.zeros((16,), jnp.float32)
        return t
    lax.fori_loop(0, _ZB, _z, 0)

    def _zb(b, t):
        pltpu.sync_copy(zbuf, agg.at[pl.ds(b * _ZB, _ZB)])
        return t
    lax.fori_loop(blo, bhi, _zb, 0)

    def _stage_idx(n, b):
        base = wid * _EPT + n * _C
        pltpu.async_copy(src_hbm.at[pl.ds(base, _C)], sidx[b], isem[b])
        pltpu.async_copy(dst_hbm.at[pl.ds(base, _C)], didx[b], isem[b])

    def _wait_idx(n, b):
        base = wid * _EPT + n * _C
        pltpu.make_async_copy(src_hbm.at[pl.ds(base, _C)], sidx[b],
                              isem[b]).wait()
        pltpu.make_async_copy(dst_hbm.at[pl.ds(base, _C)], didx[b],
                              isem[b]).wait()

    def _stage_data(n, b):
        base = wid * _EPT + n * _C
        pltpu.async_copy(h_hbm.at[sidx[b]], rows[b], dsem[b])
        pltpu.async_copy(e_hbm.at[pl.ds(base, _C)], ebuf[b], dsem[b])

    def _wait_data(n, b):
        pltpu.make_async_copy(h_hbm.at[sidx[b]], rows[b], dsem[b]).wait()
        pltpu.make_async_copy(e_hbm.at[pl.ds(wid * _EPT + n * _C, _C)],
                              ebuf[b], dsem[b]).wait()

    def _wait_scatter(b):
        pltpu.make_async_copy(rows[b], agg.at[didx[b]], ssem[b]).wait()

    def _process(n, b):
        _wait_data(n, b)

        def _v(r, u):
            for j in range(_D // 16):
                sl = pl.ds(j * 16, 16)
                rows[b][r, sl] = jnp.maximum(rows[b][r, sl] + ebuf[b][r, sl],
                                             0.0)
            return u
        lax.fori_loop(0, _C, _v, 0)
        pltpu.async_copy(rows[b], agg.at[didx[b]], ssem[b], add=True)

    plsc.subcore_barrier()

    _stage_idx(0, 0)
    _stage_idx(1, 1)
    _wait_idx(0, 0)
    _stage_data(0, 0)

    def _main(i, t):
        for db in range(_NBUF):
            n = i * _NBUF + db
            b2 = (db + 2) % _NBUF
            b1 = (db + 1) % _NBUF

            @pl.when(n + 2 < _NCH)
            def _():
                _stage_idx(n + 2, b2)

            @pl.when((n + 1 < _NCH) & (n >= 2))
            def _():
                _wait_scatter(b1)

            @pl.when(n + 1 < _NCH)
            def _():
                _wait_idx(n + 1, b1)
                _stage_data(n + 1, b1)

            _process(n, db)
        return t
    lax.fori_loop(0, _NCH // _NBUF, _main, 0)
    for n in range(_NCH - _NCH % _NBUF, _NCH):
        _process(n, n % _NBUF)
    for b in range(_NBUF):
        _wait_scatter(b)

    plsc.subcore_barrier()

    def _wb(b, t):
        pltpu.sync_copy(agg.at[pl.ds(b * _RB, _RB)],
                        out_hbm.at[pl.ds(c * _N + b * _RB, _RB)])
        return t
    lax.fori_loop(blo // (_RB // _ZB), bhi // (_RB // _ZB), _wb, 0)


def _mm_bias_body(a_ref, w_ref, b_ref, o_ref):
    o_ref[...] = (jnp.dot(a_ref[...], w_ref[...],
                          preferred_element_type=jnp.float32) + b_ref[...])


def _mm_bias(a, w, b, blk):
    m, k = a.shape
    n = w.shape[1]
    return pl.pallas_call(
        _mm_bias_body,
        grid=(m // blk,),
        in_specs=[pl.BlockSpec((blk, k), lambda i: (i, 0)),
                  pl.BlockSpec((k, n), lambda i: (0, 0)),
                  pl.BlockSpec((1, n), lambda i: (0, 0))],
        out_specs=pl.BlockSpec((blk, n), lambda i: (i, 0)),
        out_shape=jax.ShapeDtypeStruct((m, n), jnp.float32),
    )(a, w, b.reshape(1, n))


def _mlp_body(h_ref, p_ref, w1_ref, b1_ref, w2_ref, b2_ref, gs_ref, bt_ref,
              eps_ref, o_ref):
    h = h_ref[...]
    a = h * (1.0 + eps_ref[0, 0]) + p_ref[0] + p_ref[1]
    t = jnp.maximum(jnp.dot(a, w1_ref[...],
                            preferred_element_type=jnp.float32) + b1_ref[...],
                    0.0)
    t = jnp.dot(t, w2_ref[...], preferred_element_type=jnp.float32) + b2_ref[...]
    t = t * gs_ref[...] + bt_ref[...]
    o_ref[...] = jnp.maximum(t, 0.0) + h


def _mlp(h, parts, w1, b1, w2, b2, gscale, beta, eps, blk=2000):
    full = lambda i: (0, 0)
    return pl.pallas_call(
        _mlp_body,
        grid=(_N // blk,),
        in_specs=[pl.BlockSpec((blk, _D), lambda i: (i, 0)),
                  pl.BlockSpec((2, blk, _D), lambda i: (0, i, 0)),
                  pl.BlockSpec((_D, _D), full),
                  pl.BlockSpec((1, _D), full),
                  pl.BlockSpec((_D, _D), full),
                  pl.BlockSpec((1, _D), full),
                  pl.BlockSpec((1, _D), full),
                  pl.BlockSpec((1, _D), full),
                  pl.BlockSpec((1, 1), full)],
        out_specs=pl.BlockSpec((blk, _D), lambda i: (i, 0)),
        out_shape=jax.ShapeDtypeStruct((_N, _D), jnp.float32),
    )(h, parts, w1, b1.reshape(1, _D), w2, b2.reshape(1, _D),
      gscale.reshape(1, _D), beta.reshape(1, _D), eps.reshape(1, 1))


def _head_body(h_ref, b_ref, gx_ref, wg_ref, bg_ref, wc_ref, bc_ref, o_ref):
    g_iota = lax.broadcasted_iota(jnp.int32, (1, 16), 1)
    oh = (b_ref[...] == g_iota).astype(jnp.float32)
    dn = (((0,), (0,)), ((), ()))
    sums = lax.dot_general(oh, h_ref[...], dn,
                           preferred_element_type=jnp.float32)
    ones = jnp.ones((_N, 1), jnp.float32)
    counts = lax.dot_general(oh, ones, dn,
                             preferred_element_type=jnp.float32)
    pooled = sums / jnp.maximum(counts, 1.0)
    g = jnp.maximum(jnp.dot(gx_ref[...], wg_ref[...],
                            preferred_element_type=jnp.float32) + bg_ref[...],
                    0.0)
    out = (jnp.dot(pooled, wc_ref[0:_D, :],
                   preferred_element_type=jnp.float32)
           + jnp.dot(g, wc_ref[_D:_D + 32, :],
                     preferred_element_type=jnp.float32)
           + bc_ref[...])
    o_ref[...] = out


def _head(h, batch2d, gx, wg, bg, wc, bc):
    return pl.pallas_call(
        _head_body,
        out_shape=jax.ShapeDtypeStruct((16, _D), jnp.float32),
    )(h, batch2d, gx, wg, bg.reshape(1, -1), wc, bc.reshape(1, -1))


def kernel(x, edge_index, edge_attr, batch, global_x, params):
    src = edge_index[0]
    dst = edge_index[1]
    h = _mm_bias(x, params['W_in'], params['b_in'], 2000)
    e = _mm_bias(edge_attr, params['W_e'], params['b_e'], 8000)
    bn_scale = 1.0 / jnp.sqrt(jnp.float32(1.0 + 1e-5))
    for i in range(3):
        p = params['layers'][i]
        parts = _sc_edge(h, e, src, dst).reshape(2, _N, _D)
        h = _mlp(h, parts, p['W1'], p['b1'], p['W2'], p['b2'],
                 p['gamma'] * bn_scale, p['beta'], p['eps'])
    return _head(h, batch.reshape(_N, 1), global_x,
                 params['Wg'], params['bg'], params['Wc'], params['bc'])
